# baseline (speedup 1.0000x reference)
import functools

import jax
import jax.numpy as jnp
from jax import lax
from jax.experimental import pallas as pl
from jax.experimental.pallas import tpu as pltpu
from jax.experimental.pallas import tpu_sc as plsc

_B = 16384
_D = 128
_NC, _NS, _L = 2, 16, 16
_NW = _NC * _NS
_PW = _B // _NW
_NCH = 4
_CH = _PW // _NCH


def _sc_scores(n0, n1, w_emb, v_emb):
    mesh = plsc.VectorSubcoreMesh(core_axis_name="c", subcore_axis_name="s")

    @functools.partial(
        pl.kernel,
        out_type=jax.ShapeDtypeStruct((_B,), jnp.float32),
        mesh=mesh,
        compiler_params=pltpu.CompilerParams(needs_layout_passes=False),
        scratch_types=[
            pltpu.VMEM((_NCH, _CH), jnp.int32),
            pltpu.VMEM((_NCH, _CH), jnp.int32),
            pltpu.VMEM((_CH, _D), jnp.float32),
            pltpu.VMEM((_CH, _D), jnp.float32),
            pltpu.VMEM((_CH, _D), jnp.float32),
            pltpu.VMEM((_CH, _D), jnp.float32),
            pltpu.VMEM((_PW,), jnp.float32),
            pltpu.SemaphoreType.DMA,
            pltpu.SemaphoreType.DMA,
            pltpu.SemaphoreType.DMA,
            pltpu.SemaphoreType.DMA,
        ],
    )
    def k(n0h, n1h, wh, vh, outh,
          idx0, idx1, wA, wB, vA, vB, sc_v, swA, swB, svA, svB):
        wid = lax.axis_index("s") * _NC + lax.axis_index("c")
        base = wid * _PW
        pltpu.sync_copy(n0h.at[wid], idx0)
        pltpu.sync_copy(n1h.at[wid], idx1)
        wb, vb = [wA, wB], [vA, vB]
        sw, sv = [swA, swB], [svA, svB]
        pend = [None, None]

        def start(c):
            p = c & 1
            cw = pltpu.async_copy(wh.at[idx0.at[c]], wb[p], sw[p])
            cv = pltpu.async_copy(vh.at[idx1.at[c]], vb[p], sv[p])
            pend[p] = (cw, cv)

        start(0)
        iota16 = lax.iota(jnp.int32, _L)
        zero16 = jnp.zeros((_L,), jnp.int32)
        for c in range(_NCH):
            p = c & 1
            if c + 1 < _NCH:
                start(c + 1)
            cw, cv = pend[p]
            cw.wait()
            cv.wait()
            for g in range(_CH // _L):
                rows = iota16 + (g * _L)

                def jbody(jo, acc, _p=p, _rows=rows):
                    colb = zero16 + jo * 8
                    for u in range(8):
                        col = colb + u
                        wv = plsc.load_gather(wb[_p], [_rows, col])
                        vv = plsc.load_gather(vb[_p], [_rows, col])
                        acc = acc + wv * vv
                    return acc

                acc = lax.fori_loop(0, _D // 8, jbody,
                                    jnp.zeros((_L,), jnp.float32))
                sc_v[pl.ds(c * _CH + g * _L, _L)] = acc
        pltpu.sync_copy(sc_v, outh.at[pl.ds(base, _PW)])

    return k(n0, n1, w_emb, v_emb)


def _bce_mean(scores, labels):
    s2 = scores.reshape(_B // _D, _D)
    t2 = labels.reshape(_B // _D, _D)

    def body(s_ref, t_ref, o_ref):
        x = s_ref[...]
        t = t_ref[...]
        z = jnp.maximum(x, 0.0) - x * t + jnp.log1p(jnp.exp(-jnp.abs(x)))
        zs = jnp.sum(z, axis=1, keepdims=True)
        o_ref[...] = jnp.sum(zs, axis=0, keepdims=True) * (1.0 / _B)

    out = pl.pallas_call(
        body,
        out_shape=jax.ShapeDtypeStruct((1, 1), jnp.float32),
    )(s2, t2)
    return out[0, 0]


def kernel(nodes, labels, w_emb, v_emb):
    n0 = nodes[:, 0].reshape(_NW, _NCH, _CH)
    n1 = nodes[:, 1].reshape(_NW, _NCH, _CH)
    scores = _sc_scores(n0, n1, w_emb, v_emb)
    return _bce_mean(scores, labels)


# speedup vs baseline: 1.7702x; 1.7702x over previous
import functools

import jax
import jax.numpy as jnp
from jax import lax
from jax.experimental import pallas as pl
from jax.experimental.pallas import tpu as pltpu
from jax.experimental.pallas import tpu_sc as plsc

_B = 16384
_D = 128
_NC, _NS, _L = 2, 16, 16
_NW = _NC * _NS
_PW = _B // _NW
_NCH = 4
_CH = _PW // _NCH


def _sc_scores(n0, n1, w_emb, v_emb):
    mesh = plsc.VectorSubcoreMesh(core_axis_name="c", subcore_axis_name="s")

    @functools.partial(
        pl.kernel,
        out_type=jax.ShapeDtypeStruct((_B,), jnp.float32),
        mesh=mesh,
        compiler_params=pltpu.CompilerParams(needs_layout_passes=False),
        scratch_types=[
            pltpu.VMEM((_NCH, _CH), jnp.int32),
            pltpu.VMEM((_NCH, _CH), jnp.int32),
            pltpu.VMEM((_CH, _D), jnp.float32),
            pltpu.VMEM((_CH, _D), jnp.float32),
            pltpu.VMEM((_CH, _D), jnp.float32),
            pltpu.VMEM((_CH, _D), jnp.float32),
            pltpu.VMEM((_PW,), jnp.float32),
            pltpu.SemaphoreType.DMA,
            pltpu.SemaphoreType.DMA,
            pltpu.SemaphoreType.DMA,
            pltpu.SemaphoreType.DMA,
        ],
    )
    def k(n0h, n1h, wh, vh, outh,
          idx0, idx1, wA, wB, vA, vB, sc_v, swA, swB, svA, svB):
        wid = lax.axis_index("s") * _NC + lax.axis_index("c")
        base = wid * _PW
        pltpu.sync_copy(n0h.at[wid], idx0)
        pltpu.sync_copy(n1h.at[wid], idx1)
        wb, vb = [wA, wB], [vA, vB]
        sw, sv = [swA, swB], [svA, svB]
        pend = [None, None]

        def start(c):
            p = c & 1
            cw = pltpu.async_copy(wh.at[idx0.at[c]], wb[p], sw[p])
            cv = pltpu.async_copy(vh.at[idx1.at[c]], vb[p], sv[p])
            pend[p] = (cw, cv)

        start(0)
        iota16 = lax.iota(jnp.int32, _L)
        for c in range(_NCH):
            p = c & 1
            if c + 1 < _NCH:
                start(c + 1)
            cw, cv = pend[p]
            cw.wait()
            cv.wait()

            def gbody(it, _, _p=p, _c=c):
                base_row = it * _L
                scores = jnp.zeros((_L,), jnp.float32)
                for i in range(_L):
                    row = base_row + i
                    accA = (wb[_p][row, pl.ds(0, _L)]
                            * vb[_p][row, pl.ds(0, _L)])
                    accB = (wb[_p][row, pl.ds(_L, _L)]
                            * vb[_p][row, pl.ds(_L, _L)])
                    for u in range(2, _D // _L, 2):
                        accA = accA + (wb[_p][row, pl.ds(u * _L, _L)]
                                       * vb[_p][row, pl.ds(u * _L, _L)])
                        accB = accB + (wb[_p][row, pl.ds((u + 1) * _L, _L)]
                                       * vb[_p][row, pl.ds((u + 1) * _L, _L)])
                    s = jnp.sum(accA + accB)
                    sv = jnp.zeros((_L,), jnp.float32) + s
                    scores = jnp.where(iota16 == i, sv, scores)
                sc_v[pl.ds(_c * _CH + it * _L, _L)] = scores
                return _

            lax.fori_loop(0, _CH // _L, gbody, jnp.int32(0))
        pltpu.sync_copy(sc_v, outh.at[pl.ds(base, _PW)])

    return k(n0, n1, w_emb, v_emb)


def _bce_mean(scores, labels):
    s2 = scores.reshape(_B // _D, _D)
    t2 = labels.reshape(_B // _D, _D)

    def body(s_ref, t_ref, o_ref):
        x = s_ref[...]
        t = t_ref[...]
        z = jnp.maximum(x, 0.0) - x * t + jnp.log1p(jnp.exp(-jnp.abs(x)))
        zs = jnp.sum(z, axis=1, keepdims=True)
        o_ref[...] = jnp.sum(zs, axis=0, keepdims=True) * (1.0 / _B)

    out = pl.pallas_call(
        body,
        out_shape=jax.ShapeDtypeStruct((1, 1), jnp.float32),
    )(s2, t2)
    return out[0, 0]


def kernel(nodes, labels, w_emb, v_emb):
    n0 = nodes[:, 0].reshape(_NW, _NCH, _CH)
    n1 = nodes[:, 1].reshape(_NW, _NCH, _CH)
    scores = _sc_scores(n0, n1, w_emb, v_emb)
    return _bce_mean(scores, labels)


# speedup vs baseline: 2.3880x; 1.3490x over previous
import functools

import jax
import jax.numpy as jnp
from jax import lax
from jax.experimental import pallas as pl
from jax.experimental.pallas import tpu as pltpu
from jax.experimental.pallas import tpu_sc as plsc

_B = 16384
_D = 128
_NC, _NS, _L = 2, 16, 16
_NW = _NC * _NS
_PW = _B // _NW
_NCH = 4
_CH = _PW // _NCH


def _sc_scores(n0, n1, w_emb, v_emb):
    mesh = plsc.VectorSubcoreMesh(core_axis_name="c", subcore_axis_name="s")

    @functools.partial(
        pl.kernel,
        out_type=jax.ShapeDtypeStruct((_B,), jnp.float32),
        mesh=mesh,
        compiler_params=pltpu.CompilerParams(needs_layout_passes=False),
        scratch_types=[
            pltpu.VMEM((_NCH, _CH), jnp.int32),
            pltpu.VMEM((_NCH, _CH), jnp.int32),
            pltpu.VMEM((_CH, _D), jnp.float32),
            pltpu.VMEM((_CH, _D), jnp.float32),
            pltpu.VMEM((_CH, _D), jnp.float32),
            pltpu.VMEM((_CH, _D), jnp.float32),
            pltpu.VMEM((_PW,), jnp.float32),
            pltpu.VMEM((_L * 17,), jnp.float32),
            pltpu.SemaphoreType.DMA,
            pltpu.SemaphoreType.DMA,
            pltpu.SemaphoreType.DMA,
            pltpu.SemaphoreType.DMA,
        ],
    )
    def k(n0h, n1h, wh, vh, outh,
          idx0, idx1, wA, wB, vA, vB, sc_v, stg, swA, swB, svA, svB):
        wid = lax.axis_index("s") * _NC + lax.axis_index("c")
        base = wid * _PW
        pltpu.sync_copy(n0h.at[wid], idx0)
        pltpu.sync_copy(n1h.at[wid], idx1)
        wb, vb = [wA, wB], [vA, vB]
        sw, sv = [swA, swB], [svA, svB]
        pend = [None, None]

        def start(c):
            p = c & 1
            cw = pltpu.async_copy(wh.at[idx0.at[c]], wb[p], sw[p])
            cv = pltpu.async_copy(vh.at[idx1.at[c]], vb[p], sv[p])
            pend[p] = (cw, cv)

        start(0)
        iota16 = lax.iota(jnp.int32, _L)
        for c in range(_NCH):
            p = c & 1
            if c + 1 < _NCH:
                start(c + 1)
            cw, cv = pend[p]
            cw.wait()
            cv.wait()

            def gbody(it, _, _p=p, _c=c):
                base_row = it * _L
                for i in range(_L):
                    row = base_row + i
                    ps = [wb[_p][row, pl.ds(u * _L, _L)]
                          * vb[_p][row, pl.ds(u * _L, _L)]
                          for u in range(_D // _L)]
                    while len(ps) > 1:
                        ps = [a + b for a, b in zip(ps[::2], ps[1::2])]
                    plsc.store_scatter(stg, [iota16 * 17 + i], ps[0])
                acc = stg[pl.ds(0, _L)]
                for j in range(1, _L):
                    acc = acc + stg[pl.ds(j * 17, _L)]
                sc_v[pl.ds(_c * _CH + it * _L, _L)] = acc
                return _

            lax.fori_loop(0, _CH // _L, gbody, jnp.int32(0))
        pltpu.sync_copy(sc_v, outh.at[pl.ds(base, _PW)])

    return k(n0, n1, w_emb, v_emb)


def _bce_mean(scores, labels):
    s2 = scores.reshape(_B // _D, _D)
    t2 = labels.reshape(_B // _D, _D)

    def body(s_ref, t_ref, o_ref):
        x = s_ref[...]
        t = t_ref[...]
        z = jnp.maximum(x, 0.0) - x * t + jnp.log1p(jnp.exp(-jnp.abs(x)))
        zs = jnp.sum(z, axis=1, keepdims=True)
        o_ref[...] = jnp.sum(zs, axis=0, keepdims=True) * (1.0 / _B)

    out = pl.pallas_call(
        body,
        out_shape=jax.ShapeDtypeStruct((1, 1), jnp.float32),
    )(s2, t2)
    return out[0, 0]


def kernel(nodes, labels, w_emb, v_emb):
    n0 = nodes[:, 0].reshape(_NW, _NCH, _CH)
    n1 = nodes[:, 1].reshape(_NW, _NCH, _CH)
    scores = _sc_scores(n0, n1, w_emb, v_emb)
    return _bce_mean(scores, labels)


# speedup vs baseline: 2.5013x; 1.0474x over previous
import functools

import jax
import jax.numpy as jnp
from jax import lax
from jax.experimental import pallas as pl
from jax.experimental.pallas import tpu as pltpu
from jax.experimental.pallas import tpu_sc as plsc

_B = 16384
_D = 128
_NC, _NS, _L = 2, 16, 16
_NW = _NC * _NS
_PW = _B // _NW
_NCH = 4
_CH = _PW // _NCH


def _sc_scores(n0, n1, w_emb, v_emb):
    mesh = plsc.VectorSubcoreMesh(core_axis_name="c", subcore_axis_name="s")

    @functools.partial(
        pl.kernel,
        out_type=jax.ShapeDtypeStruct((_B,), jnp.float32),
        mesh=mesh,
        compiler_params=pltpu.CompilerParams(needs_layout_passes=False),
        scratch_types=[
            pltpu.VMEM((_NCH, _CH), jnp.int32),
            pltpu.VMEM((_NCH, _CH), jnp.int32),
            pltpu.VMEM((_CH, _D), jnp.float32),
            pltpu.VMEM((_CH, _D), jnp.float32),
            pltpu.VMEM((_CH, _D), jnp.float32),
            pltpu.VMEM((_CH, _D), jnp.float32),
            pltpu.VMEM((_PW,), jnp.float32),
            pltpu.VMEM(((_CH // _L) * _L * 17,), jnp.float32),
            pltpu.SemaphoreType.DMA,
            pltpu.SemaphoreType.DMA,
            pltpu.SemaphoreType.DMA,
            pltpu.SemaphoreType.DMA,
        ],
    )
    def k(n0h, n1h, wh, vh, outh,
          idx0, idx1, wA, wB, vA, vB, sc_v, stg, swA, swB, svA, svB):
        wid = lax.axis_index("s") * _NC + lax.axis_index("c")
        base = wid * _PW
        pltpu.sync_copy(n0h.at[wid], idx0)
        pltpu.sync_copy(n1h.at[wid], idx1)
        wb, vb = [wA, wB], [vA, vB]
        sw, sv = [swA, swB], [svA, svB]
        pend = [None, None]

        def start(c):
            p = c & 1
            cw = pltpu.async_copy(wh.at[idx0.at[c]], wb[p], sw[p])
            cv = pltpu.async_copy(vh.at[idx1.at[c]], vb[p], sv[p])
            pend[p] = (cw, cv)

        start(0)
        iota16 = lax.iota(jnp.int32, _L)
        for c in range(_NCH):
            p = c & 1
            if c + 1 < _NCH:
                start(c + 1)
            cw, cv = pend[p]
            cw.wait()
            cv.wait()

            @plsc.parallel_loop(0, _CH // _L)
            def gbody(it, _p=p, _c=c):
                base_row = it * _L
                off = it * (_L * 17)

                def load_pair(row):
                    vals = []
                    for u in range(_D // _L):
                        vals.append(wb[_p][row, pl.ds(u * _L, _L)])
                        vals.append(vb[_p][row, pl.ds(u * _L, _L)])
                    return vals

                def reduce_store(vals, i):
                    ps = [vals[2 * u] * vals[2 * u + 1]
                          for u in range(_D // _L)]
                    while len(ps) > 1:
                        ps = [a + b for a, b in zip(ps[::2], ps[1::2])]
                    plsc.store_scatter(stg, [iota16 * 17 + (off + i)], ps[0])

                vals = load_pair(base_row)
                for i in range(_L):
                    nxt = load_pair(base_row + i + 1) if i + 1 < _L else None
                    reduce_store(vals, i)
                    vals = nxt
                qs = [stg[pl.ds(off + j * 17, _L)] for j in range(_L)]
                while len(qs) > 1:
                    qs = [a + b for a, b in zip(qs[::2], qs[1::2])]
                sc_v[pl.ds(_c * _CH + it * _L, _L)] = qs[0]
        pltpu.sync_copy(sc_v, outh.at[pl.ds(base, _PW)])

    return k(n0, n1, w_emb, v_emb)


def _bce_mean(scores, labels):
    s2 = scores.reshape(_B // _D, _D)
    t2 = labels.reshape(_B // _D, _D)

    def body(s_ref, t_ref, o_ref):
        x = s_ref[...]
        t = t_ref[...]
        z = jnp.maximum(x, 0.0) - x * t + jnp.log1p(jnp.exp(-jnp.abs(x)))
        zs = jnp.sum(z, axis=1, keepdims=True)
        o_ref[...] = jnp.sum(zs, axis=0, keepdims=True) * (1.0 / _B)

    out = pl.pallas_call(
        body,
        out_shape=jax.ShapeDtypeStruct((1, 1), jnp.float32),
    )(s2, t2)
    return out[0, 0]


def kernel(nodes, labels, w_emb, v_emb):
    n0 = nodes[:, 0].reshape(_NW, _NCH, _CH)
    n1 = nodes[:, 1].reshape(_NW, _NCH, _CH)
    scores = _sc_scores(n0, n1, w_emb, v_emb)
    return _bce_mean(scores, labels)
